# baseline (speedup 1.0000x reference)
import functools

import jax
import jax.numpy as jnp
from jax import lax
from jax.experimental import pallas as pl
from jax.experimental.pallas import tpu as pltpu
from jax.experimental.pallas import tpu_sc as plsc

S = 200
G = 500
NE = 397
D = 64
N = 400000
E = 800000

_BN0 = 3200


def _emb_kernel(ids_ref, emb_ref, o_ref):
    ids = ids_ref[...]
    iota = lax.broadcasted_iota(jnp.int32, (_BN0, 256), 1)
    onehot = (ids == iota).astype(jnp.float32)
    o_ref[...] = jnp.dot(onehot, emb_ref[...], preferred_element_type=jnp.float32)


def _emb_lookup(emb_pad, ids2d):
    return pl.pallas_call(
        _emb_kernel,
        grid=(N // _BN0,),
        in_specs=[
            pl.BlockSpec((_BN0, 1), lambda i: (i, 0)),
            pl.BlockSpec((256, D), lambda i: (0, 0)),
        ],
        out_specs=pl.BlockSpec((_BN0, D), lambda i: (i, 0)),
        out_shape=jax.ShapeDtypeStruct((N, D), jnp.float32),
    )(ids2d, emb_pad)


_BN2 = 3200


def _mlp_kernel(x_ref, a_ref, sc_ref, w1_ref, b1_ref, w2_ref, b2_ref, g_ref,
                be_ref, o_ref):
    x = x_ref[...]
    h = sc_ref[0, 0] * x + a_ref[...]
    h = jnp.maximum(jnp.dot(h, w1_ref[...], preferred_element_type=jnp.float32)
                    + b1_ref[...], 0.0)
    h = jnp.dot(h, w2_ref[...], preferred_element_type=jnp.float32) + b2_ref[...]
    y = x + h
    m = y.mean(-1, keepdims=True)
    v = ((y - m) ** 2).mean(-1, keepdims=True)
    o_ref[...] = (y - m) / jnp.sqrt(v + 1e-5) * g_ref[...] + be_ref[...]


def _mlp_aug_kernel(x_ref, a_ref, val_ref, sc_ref, w1_ref, b1_ref, w2_ref,
                    b2_ref, g_ref, be_ref, o_ref):
    x = x_ref[...]
    h = sc_ref[0, 0] * x + a_ref[...]
    h = jnp.maximum(jnp.dot(h, w1_ref[...], preferred_element_type=jnp.float32)
                    + b1_ref[...], 0.0)
    h = jnp.dot(h, w2_ref[...], preferred_element_type=jnp.float32) + b2_ref[...]
    y = x + h
    m = y.mean(-1, keepdims=True)
    v = ((y - m) ** 2).mean(-1, keepdims=True)
    y = (y - m) / jnp.sqrt(v + 1e-5) * g_ref[...] + be_ref[...]
    val = val_ref[...]
    o_ref[...] = jnp.concatenate(
        [y * val, val, jnp.zeros((_BN2, 15), jnp.float32)], axis=1)


def _wspecs():
    return [
        pl.BlockSpec((1, 128), lambda i: (0, 0)),
        pl.BlockSpec((D, D), lambda i: (0, 0)),
        pl.BlockSpec((1, D), lambda i: (0, 0)),
        pl.BlockSpec((D, D), lambda i: (0, 0)),
        pl.BlockSpec((1, D), lambda i: (0, 0)),
        pl.BlockSpec((1, D), lambda i: (0, 0)),
        pl.BlockSpec((1, D), lambda i: (0, 0)),
    ]


def _mlp(x, agg, scale, W1, b1, W2, b2, g, be):
    return pl.pallas_call(
        _mlp_kernel,
        grid=(N // _BN2,),
        in_specs=[
            pl.BlockSpec((_BN2, D), lambda i: (i, 0)),
            pl.BlockSpec((_BN2, D), lambda i: (i, 0)),
        ] + _wspecs(),
        out_specs=pl.BlockSpec((_BN2, D), lambda i: (i, 0)),
        out_shape=jax.ShapeDtypeStruct((N, D), jnp.float32),
    )(x, agg, scale, W1, b1, W2, b2, g, be)


_NP4 = 409600


def _mlp_aug(x, agg, valid2d, scale, W1, b1, W2, b2, g, be):
    return pl.pallas_call(
        _mlp_aug_kernel,
        grid=(N // _BN2,),
        in_specs=[
            pl.BlockSpec((_BN2, D), lambda i: (i, 0)),
            pl.BlockSpec((_BN2, D), lambda i: (i, 0)),
            pl.BlockSpec((_BN2, 1), lambda i: (i, 0)),
        ] + _wspecs(),
        out_specs=pl.BlockSpec((_BN2, 80), lambda i: (i, 0)),
        out_shape=jax.ShapeDtypeStruct((_NP4, 80), jnp.float32),
    )(x, agg, valid2d, scale, W1, b1, W2, b2, g, be)


_EPAD = 819200
_RNG = 25600
_NPAD3 = 16 * _RNG
_TRASH = 25600
_ECHUNK_ROWS = 16
_NCHUNK = 25
_WB = 1600


def _edge_scatter(x, src2d, dst2d, zeros64):
    mesh = plsc.VectorSubcoreMesh(core_axis_name="c", subcore_axis_name="s")

    @functools.partial(
        pl.kernel,
        out_type=jax.ShapeDtypeStruct((_NPAD3, D), jnp.float32),
        mesh=mesh,
        compiler_params=pltpu.CompilerParams(use_tc_tiling_on_sc=False),
        scratch_types=[
            pltpu.VMEM((_ECHUNK_ROWS, 128), jnp.int32),
            pltpu.VMEM((_ECHUNK_ROWS, 128), jnp.int32),
            pltpu.VMEM((_ECHUNK_ROWS, 128), jnp.int32),
            pltpu.VMEM((128, D), jnp.float32),
            pltpu.VMEM_SHARED((_RNG + 8, D), jnp.float32),
            pltpu.SemaphoreType.DMA,
        ],
    )
    def k(x_hbm, src_hbm, dst_hbm, z_hbm, agg_hbm, srcb, dstb, ldstb, rows,
          acc, sem):
        c = lax.axis_index("c")
        t = lax.axis_index("s")
        tile_row0 = t * (_NCHUNK * _ECHUNK_ROWS)

        def per_range(p, carry):
            base = (c * 8 + p) * _RNG
            pltpu.sync_copy(z_hbm.at[pl.ds(0, _WB)],
                            acc.at[pl.ds(t * _WB, _WB)])
            plsc.subcore_barrier()

            def per_chunk(ch, carry2):
                row0 = tile_row0 + ch * _ECHUNK_ROWS
                pltpu.sync_copy(src_hbm.at[pl.ds(row0, _ECHUNK_ROWS)], srcb)
                pltpu.sync_copy(dst_hbm.at[pl.ds(row0, _ECHUNK_ROWS)], dstb)

                def per_group(q, carry3):
                    for h in range(8):
                        dd = dstb[q, pl.ds(h * 16, 16)]
                        ld = dd - base
                        msk = (ld >= 0) & (ld < _RNG)
                        ldstb[q, pl.ds(h * 16, 16)] = jnp.where(msk, ld, _TRASH)
                    return carry3

                lax.fori_loop(0, _ECHUNK_ROWS, per_group, 0)

                def per_sub(j, carry3):
                    pltpu.async_copy(x_hbm.at[srcb.at[j]], rows, sem).wait()
                    pltpu.sync_copy(rows, acc.at[ldstb.at[j]], add=True)
                    return carry3

                lax.fori_loop(0, _ECHUNK_ROWS, per_sub, 0)
                return carry2

            lax.fori_loop(0, _NCHUNK, per_chunk, 0)
            plsc.subcore_barrier()
            pltpu.sync_copy(acc.at[pl.ds(t * _WB, _WB)],
                            agg_hbm.at[pl.ds(base + t * _WB, _WB)])
            plsc.subcore_barrier()
            return carry

        lax.fori_loop(0, 8, per_range, 0)

    return k(x, src2d, dst2d, zeros64)


_SEG = G * S
_RNGP = 12544
_SEGPAD = 8 * _RNGP
_TRASHP = 12544
_PCHUNK_ROWS = 8
_NCHUNKP = 25
_WBP = 784


def _seg_pool(x_aug, seg2d, zeros80):
    mesh = plsc.VectorSubcoreMesh(core_axis_name="c", subcore_axis_name="s")

    @functools.partial(
        pl.kernel,
        out_type=jax.ShapeDtypeStruct((_SEGPAD, 80), jnp.float32),
        mesh=mesh,
        compiler_params=pltpu.CompilerParams(use_tc_tiling_on_sc=False),
        scratch_types=[
            pltpu.VMEM((_PCHUNK_ROWS, 128), jnp.int32),
            pltpu.VMEM((_PCHUNK_ROWS, 128), jnp.int32),
            pltpu.VMEM((128, 80), jnp.float32),
            pltpu.VMEM_SHARED((_RNGP + 8, 80), jnp.float32),
            pltpu.SemaphoreType.DMA,
        ],
    )
    def k(x_hbm, seg_hbm, z_hbm, out_hbm, segb, lsegb, rows, acc, sem):
        c = lax.axis_index("c")
        t = lax.axis_index("s")
        tile_row0 = t * (_NCHUNKP * _PCHUNK_ROWS)

        def per_range(p, carry):
            base = (c * 4 + p) * _RNGP
            pltpu.sync_copy(z_hbm.at[pl.ds(0, _WBP)],
                            acc.at[pl.ds(t * _WBP, _WBP)])
            plsc.subcore_barrier()

            def per_chunk(ch, carry2):
                row0 = tile_row0 + ch * _PCHUNK_ROWS
                pltpu.sync_copy(seg_hbm.at[pl.ds(row0, _PCHUNK_ROWS)], segb)

                def per_group(q, carry3):
                    for h in range(8):
                        ss = segb[q, pl.ds(h * 16, 16)]
                        ls = ss - base
                        msk = (ls >= 0) & (ls < _RNGP)
                        lsegb[q, pl.ds(h * 16, 16)] = jnp.where(msk, ls, _TRASHP)
                    return carry3

                lax.fori_loop(0, _PCHUNK_ROWS, per_group, 0)

                def per_sub(j, carry3):
                    off = (row0 + j) * 128
                    pltpu.sync_copy(x_hbm.at[pl.ds(off, 128)], rows)
                    pltpu.sync_copy(rows, acc.at[lsegb.at[j]], add=True)
                    return carry3

                lax.fori_loop(0, _PCHUNK_ROWS, per_sub, 0)
                return carry2

            lax.fori_loop(0, _NCHUNKP, per_chunk, 0)
            plsc.subcore_barrier()
            pltpu.sync_copy(acc.at[pl.ds(t * _WBP, _WBP)],
                            out_hbm.at[pl.ds(base + t * _WBP, _WBP)])
            plsc.subcore_barrier()
            return carry

        lax.fori_loop(0, 4, per_range, 0)

    return k(x_aug, seg2d, zeros80)


_GB = 10


def _final_kernel(sums_ref, cnts_ref, m_ref, o_ref, s1_ref, s2_ref, nv_ref):
    gi = pl.program_id(0)

    @pl.when(gi == 0)
    def _():
        s1_ref[...] = jnp.zeros_like(s1_ref)
        s2_ref[...] = jnp.zeros_like(s2_ref)
        nv_ref[...] = jnp.zeros_like(nv_ref)

    mm = m_ref[...]
    for i in range(_GB):
        sums_i = sums_ref[i]
        cnts_i = cnts_ref[i]
        present = (cnts_i > 0.0).astype(jnp.float32)
        pooled = sums_i / jnp.maximum(cnts_i, 1.0)
        weighted = pooled * present
        cs = jnp.dot(mm, weighted, preferred_element_type=jnp.float32)
        cc = jnp.dot(mm, present, preferred_element_type=jnp.float32)
        gt = cs / jnp.maximum(cc, 1.0)
        gv = (cc > 0.0).astype(jnp.float32)
        s1_ref[...] += gt * gv
        s2_ref[...] += gt * gt * gv
        nv_ref[...] += jnp.broadcast_to(gv, (400, D))

    @pl.when(gi == pl.num_programs(0) - 1)
    def _():
        nv = nv_ref[...]
        mean = s1_ref[...] / jnp.maximum(nv, 1.0)
        var = (s2_ref[...] - nv * mean * mean) / jnp.maximum(nv - 1.0, 1.0)
        var = jnp.maximum(var, 0.0)
        std = jnp.where(nv > 1.0, jnp.sqrt(var), 0.0)
        o_ref[...] = jnp.concatenate([mean, std], axis=1)


def _finalize(sums3d, cnts3d, m_pad):
    return pl.pallas_call(
        _final_kernel,
        grid=(G // _GB,),
        in_specs=[
            pl.BlockSpec((_GB, S, D), lambda i: (i, 0, 0)),
            pl.BlockSpec((_GB, S, 1), lambda i: (i, 0, 0)),
            pl.BlockSpec((400, S), lambda i: (0, 0)),
        ],
        out_specs=pl.BlockSpec((400, 2 * D), lambda i: (0, 0)),
        out_shape=jax.ShapeDtypeStruct((400, 2 * D), jnp.float32),
        scratch_shapes=[
            pltpu.VMEM((400, D), jnp.float32),
            pltpu.VMEM((400, D), jnp.float32),
            pltpu.VMEM((400, D), jnp.float32),
        ],
    )(sums3d, cnts3d, m_pad)


def kernel(species_emb, eps0, W1_0, b1_0, W2_0, b2_0, g0, be0, eps1, W1_1,
           b1_1, W2_1, b2_1, g1, be1, edge_index, sp_ids, leaf_mask, tree_ids,
           clade_membership, n_edges):
    sp_ids = sp_ids.astype(jnp.int32)
    tree_ids = tree_ids.astype(jnp.int32)
    emb_ids = jnp.where((sp_ids < 0) | (sp_ids >= S), S, sp_ids)
    ids2d = emb_ids.reshape(N, 1)
    emb_pad = jnp.zeros((256, D), jnp.float32).at[: S + 1].set(species_emb)

    src = edge_index[0].astype(jnp.int32)
    dst = edge_index[1].astype(jnp.int32)
    pad_e = _EPAD - E
    src2d = jnp.concatenate([src, jnp.zeros((pad_e,), jnp.int32)]).reshape(-1, 128)
    dst2d = jnp.concatenate(
        [dst, jnp.full((pad_e,), jnp.int32(1 << 28))]).reshape(-1, 128)
    zeros64 = jnp.zeros((1600, D), jnp.float32)
    zeros80 = jnp.zeros((800, 80), jnp.float32)

    valid = (leaf_mask & (sp_ids >= 0) & (sp_ids < S)).astype(jnp.float32)
    valid2d = valid.reshape(N, 1)
    seg = tree_ids * S + jnp.clip(sp_ids, 0, S - 1)
    seg2d = jnp.concatenate(
        [seg, jnp.full((_NP4 - N,), jnp.int32(1 << 28))]).reshape(-1, 128)

    sc0 = jnp.full((1, 128), 1.0 + eps0, jnp.float32)
    sc1 = jnp.full((1, 128), 1.0 + eps1, jnp.float32)

    x0 = _emb_lookup(emb_pad, ids2d)
    agg0 = _edge_scatter(x0, src2d, dst2d, zeros64)[:N]
    x1 = _mlp(x0, agg0, sc0, W1_0, b1_0.reshape(1, D), W2_0,
              b2_0.reshape(1, D), g0.reshape(1, D), be0.reshape(1, D))
    agg1 = _edge_scatter(x1, src2d, dst2d, zeros64)[:N]
    x_aug = _mlp_aug(x1, agg1, valid2d, sc1, W1_1, b1_1.reshape(1, D), W2_1,
                     b2_1.reshape(1, D), g1.reshape(1, D), be1.reshape(1, D))

    sums_aug = _seg_pool(x_aug, seg2d, zeros80)[:_SEG]
    sums3d = sums_aug[:, :D].reshape(G, S, D)
    cnts3d = sums_aug[:, D].reshape(G, S, 1)

    m_pad = jnp.zeros((400, S), jnp.float32).at[:NE].set(
        clade_membership.astype(jnp.float32))
    out = _finalize(sums3d, cnts3d, m_pad)
    return out[:NE]
